# baseline (speedup 1.0000x reference)
import jax
import jax.numpy as jnp
from jax.experimental import pallas as pl
from jax.experimental.pallas import tpu as pltpu

L, B, H, DK, DV = 2048, 128, 32, 128, 128
LT = 128
NL = L // LT


def _qenc_body(q_ref, wq_ref, bq_ref, o_ref):
    o_ref[...] = (q_ref[...] @ wq_ref[...] + bq_ref[...])


def _attn_body(q_ref, k_ref, v_ref, r_ref, o_ref, acc_ref, m_ref, l_ref):
    i = pl.program_id(0)

    @pl.when(i == 0)
    def _():
        m_ref[...] = jnp.full_like(m_ref, -jnp.inf)
        l_ref[...] = jnp.zeros_like(l_ref)
        acc_ref[...] = jnp.zeros_like(acc_ref)

    for b in range(B):
        lo, hi = b * H, (b + 1) * H
        kb = k_ref[:, b, :]
        vb = v_ref[:, b, :]
        qb = q_ref[lo:hi]
        s = jax.lax.dot_general(qb, kb, (((1,), (1,)), ((), ())),
                                preferred_element_type=jnp.float32)
        s = s * r_ref[b]
        mo = m_ref[lo:hi]
        mn = jnp.maximum(mo, jnp.max(s, axis=1, keepdims=True))
        alpha = jnp.exp(mo - mn)
        e = jnp.exp(s - mn)
        l_ref[lo:hi] = alpha * l_ref[lo:hi] + jnp.sum(e, axis=1,
                                                      keepdims=True)
        pv = jax.lax.dot_general(e, vb, (((1,), (0,)), ((), ())),
                                 preferred_element_type=jnp.float32)
        acc_ref[lo:hi] = alpha * acc_ref[lo:hi] + pv
        m_ref[lo:hi] = mn

    @pl.when(i == NL - 1)
    def _():
        o_ref[...] = acc_ref[...] / l_ref[...]


def _agg_body(r_ref, wagg_ref, bagg_ref, o_ref):
    o_ref[...] = (r_ref[...] @ wagg_ref[...] + bagg_ref[...])


def kernel(query, keys, vals, rpe, Wq, bq, Wagg, bagg):
    rpe2 = rpe.reshape(L, B).T.reshape(B, 1, L)

    q_all = pl.pallas_call(
        _qenc_body,
        out_shape=jax.ShapeDtypeStruct((B, H * DK), jnp.float32),
    )(query, Wq, bq.reshape(1, H * DK))
    qh = q_all.reshape(B * H, DK)

    res = pl.pallas_call(
        _attn_body,
        grid=(NL,),
        in_specs=[
            pl.BlockSpec((B * H, DK), lambda i: (0, 0)),
            pl.BlockSpec((LT, B, DK), lambda i: (i, 0, 0)),
            pl.BlockSpec((LT, B, DV), lambda i: (i, 0, 0)),
            pl.BlockSpec((B, 1, LT), lambda i: (0, 0, i)),
        ],
        out_specs=pl.BlockSpec((B * H, DV), lambda i: (0, 0)),
        out_shape=jax.ShapeDtypeStruct((B * H, DV), jnp.float32),
        scratch_shapes=[
            pltpu.VMEM((B * H, DV), jnp.float32),
            pltpu.VMEM((B * H, 1), jnp.float32),
            pltpu.VMEM((B * H, 1), jnp.float32),
        ],
    )(qh, keys, vals, rpe2)

    out = pl.pallas_call(
        _agg_body,
        out_shape=jax.ShapeDtypeStruct((B, DV), jnp.float32),
    )(res.reshape(B, H * DV), Wagg, bagg.reshape(1, DV))
    return out


# speedup vs baseline: 2.2723x; 2.2723x over previous
import jax
import jax.numpy as jnp
from jax.experimental import pallas as pl
from jax.experimental.pallas import tpu as pltpu

L, B, H, DK, DV = 2048, 128, 32, 128, 128
LT = 128
NL = L // LT
G = B // 8
GH = 8 * H


def _qenc_body(q_ref, wq_ref, bq_ref, o_ref):
    o_ref[...] = (q_ref[...] @ wq_ref[...] + bq_ref[...])


def _attn_body(q_ref, k_ref, v_ref, r_ref, o_ref, acc_ref, m_ref, l_ref):
    i = pl.program_id(0)

    @pl.when(i == 0)
    def _():
        m_ref[...] = jnp.full_like(m_ref, -jnp.inf)
        l_ref[...] = jnp.zeros_like(l_ref)

    rc = r_ref[0]
    row_b = jax.lax.broadcasted_iota(jnp.int32, (LT * 8, GH), 0) % 8
    col_b = jax.lax.broadcasted_iota(jnp.int32, (LT * 8, GH), 1) // H
    diag = row_b == col_b

    for g in range(G):
        sl = slice(GH * g, GH * (g + 1))
        kg = k_ref[:, 8 * g:8 * (g + 1), :].reshape(LT * 8, DK)
        vg = v_ref[:, 8 * g:8 * (g + 1), :].reshape(LT * 8, DV)
        qg = q_ref[sl]
        s = jax.lax.dot_general(kg, qg, (((1,), (1,)), ((), ())),
                                preferred_element_type=jnp.float32)
        s = s * rc[:, g:g + 1]
        s = jnp.where(diag, s, -1e30)
        mo = m_ref[:, sl]
        mn = jnp.maximum(mo, jnp.max(s, axis=0, keepdims=True))
        alpha = jnp.exp(mo - mn)
        e = jnp.exp(s - mn)
        l_ref[:, sl] = alpha * l_ref[:, sl] + jnp.sum(e, axis=0,
                                                      keepdims=True)
        pv = jax.lax.dot_general(e, vg, (((0,), (0,)), ((), ())),
                                 preferred_element_type=jnp.float32)
        acc = alpha.reshape(GH, 1) * acc_ref[sl] + pv

        @pl.when(i == 0)
        def _():
            acc_ref[sl] = pv

        @pl.when(i > 0)
        def _():
            acc_ref[sl] = acc

        m_ref[:, sl] = mn

    @pl.when(i == NL - 1)
    def _():
        o_ref[...] = acc_ref[...] / l_ref[0].reshape(B * H, 1)


def _agg_body(r_ref, wagg_ref, bagg_ref, o_ref):
    o_ref[...] = (r_ref[...] @ wagg_ref[...] + bagg_ref[...])


def kernel(query, keys, vals, rpe, Wq, bq, Wagg, bagg):
    rc = rpe.reshape(NL, LT, G, 8).transpose(0, 1, 3, 2).reshape(
        NL, LT * 8, G)

    q_all = pl.pallas_call(
        _qenc_body,
        out_shape=jax.ShapeDtypeStruct((B, H * DK), jnp.float32),
    )(query, Wq, bq.reshape(1, H * DK))
    qh = q_all.reshape(B * H, DK)

    res = pl.pallas_call(
        _attn_body,
        grid=(NL,),
        in_specs=[
            pl.BlockSpec((B * H, DK), lambda i: (0, 0)),
            pl.BlockSpec((LT, B, DK), lambda i: (i, 0, 0)),
            pl.BlockSpec((LT, B, DV), lambda i: (i, 0, 0)),
            pl.BlockSpec((1, LT * 8, G), lambda i: (i, 0, 0)),
        ],
        out_specs=pl.BlockSpec((B * H, DV), lambda i: (0, 0)),
        out_shape=jax.ShapeDtypeStruct((B * H, DV), jnp.float32),
        scratch_shapes=[
            pltpu.VMEM((B * H, DV), jnp.float32),
            pltpu.VMEM((1, B * H), jnp.float32),
            pltpu.VMEM((1, B * H), jnp.float32),
        ],
    )(qh, keys, vals, rc)

    out = pl.pallas_call(
        _agg_body,
        out_shape=jax.ShapeDtypeStruct((B, DV), jnp.float32),
    )(res.reshape(B, H * DV), Wagg, bagg.reshape(1, DV))
    return out
